# Initial kernel scaffold; baseline (speedup 1.0000x reference)
import jax
import jax.numpy as jnp
from jax.experimental import pallas as pl


def kernel(cls_ids, bboxes, cls_embed_table):
    raise NotImplementedError("write your pallas kernel here")


# speedup vs baseline: 1.9324x; 1.9324x over previous
import jax
import jax.numpy as jnp
from jax import lax
from jax.experimental import pallas as pl
from jax.experimental.pallas import tpu as pltpu
from jax.experimental.pallas import tpu_sc as plsc

NC, NS, L = 2, 16, 16
NW = NC * NS
D_EMB = 124
D_OUT = 128
R = 128


def _body(ids_hbm, bbox_hbm, table_hbm, out_hbm, idx_v, rows_v, bbox_v, sem):
    wid = lax.axis_index("s") * NC + lax.axis_index("c")
    n_tok = ids_hbm.shape[0]
    per_w = n_tok // NW
    n_blk = per_w // R
    base = wid * per_w

    iota = lax.iota(jnp.int32, L)
    hi_mask = iota >= (L - 4)

    def blk(t, carry):
        r0 = base + t * R
        pltpu.sync_copy(ids_hbm.at[pl.ds(r0, R)], idx_v)
        gather = pltpu.async_copy(table_hbm.at[idx_v], rows_v, sem)
        pltpu.sync_copy(bbox_hbm.at[pl.ds(r0 * 4, R * 4)], bbox_v)
        gather.wait()

        def merge(g, c):
            vb = bbox_v[pl.ds(g * L, L)]
            for k in range(4):
                row = g * 4 + k
                old = rows_v[row, pl.ds(D_OUT - L, L)]
                perm = vb.at[jnp.where(hi_mask, iota - (L - 4) + 4 * k, 0)].get(
                    mode="promise_in_bounds")
                rows_v[row, pl.ds(D_OUT - L, L)] = jnp.where(hi_mask, perm, old)
            return c

        lax.fori_loop(0, R // 4, merge, 0, unroll=4)
        pltpu.sync_copy(rows_v, out_hbm.at[pl.ds(r0, R)])
        return carry

    lax.fori_loop(0, n_blk, blk, 0)


def kernel(cls_ids, bboxes, cls_embed_table):
    B, T = cls_ids.shape
    V, Dm = cls_embed_table.shape
    ids = cls_ids.reshape(-1).astype(jnp.int32)
    bbox_flat = bboxes.reshape(-1)
    table_pad = jnp.pad(cls_embed_table, ((0, 0), (0, D_OUT - Dm)))

    mesh = plsc.VectorSubcoreMesh(
        core_axis_name="c", subcore_axis_name="s", num_cores=NC, num_subcores=NS
    )
    out = pl.kernel(
        _body,
        out_type=jax.ShapeDtypeStruct((B * T, D_OUT), jnp.float32),
        mesh=mesh,
        compiler_params=pltpu.CompilerParams(use_tc_tiling_on_sc=False),
        scratch_types=[
            pltpu.VMEM((R,), jnp.int32),
            pltpu.VMEM((R, D_OUT), jnp.float32),
            pltpu.VMEM((R * 4,), jnp.float32),
            pltpu.SemaphoreType.DMA,
        ],
    )(ids, bbox_flat, table_pad)
    return out.reshape(B, T, D_OUT)


# speedup vs baseline: 2.2306x; 1.1543x over previous
import jax
import jax.numpy as jnp
from jax import lax
from jax.experimental import pallas as pl
from jax.experimental.pallas import tpu as pltpu
from jax.experimental.pallas import tpu_sc as plsc

NC, NS, L = 2, 16, 16
NW = NC * NS
D_EMB = 124
D_OUT = 128
R = 128
NB = 4


def _body(ids_hbm, bbox_hbm, table_hbm, out_hbm, idx_v, rows_v, bbox_v, *sems):
    gsems, wsems = sems[:NB], sems[NB:]
    wid = lax.axis_index("s") * NC + lax.axis_index("c")
    n_tok = ids_hbm.shape[0]
    per_w = n_tok // NW
    n_blk = per_w // R
    base = wid * per_w

    iota = lax.iota(jnp.int32, L)
    hi_mask = iota >= (L - 4)

    def merge(b):
        def grp(g, c):
            vb = bbox_v[b, pl.ds(g * L, L)]
            for k in range(4):
                row = g * 4 + k
                old = rows_v[b, row, pl.ds(D_OUT - L, L)]
                perm = vb.at[jnp.where(hi_mask, iota - (L - 4) + 4 * k, 0)].get(
                    mode="promise_in_bounds")
                rows_v[b, row, pl.ds(D_OUT - L, L)] = jnp.where(hi_mask, perm, old)
            return c
        lax.fori_loop(0, R // 4, grp, 0, unroll=4)

    def launch(blk_id, b):
        pltpu.sync_copy(ids_hbm.at[pl.ds(base + blk_id * R, R)], idx_v.at[b])
        pltpu.async_copy(table_hbm.at[idx_v.at[b]], rows_v.at[b], gsems[b])

    launch(0, 0)
    launch(1, 1)

    def super_blk(i, carry):
        for b in range(NB):
            t = i * NB + b
            b2 = (b + 2) % NB
            @pl.when(t + 2 < n_blk)
            def _():
                @pl.when(t >= 2)
                def _():
                    pltpu.make_async_copy(
                        rows_v.at[b2], out_hbm.at[pl.ds(base, R)], wsems[b2]
                    ).wait()
                launch(t + 2, b2)
            r0 = base + t * R
            pltpu.sync_copy(bbox_hbm.at[pl.ds(r0 * 4, R * 4)], bbox_v.at[b])
            pltpu.make_async_copy(
                table_hbm.at[idx_v.at[b]], rows_v.at[b], gsems[b]
            ).wait()
            merge(b)
            pltpu.async_copy(rows_v.at[b], out_hbm.at[pl.ds(r0, R)], wsems[b])
        return carry

    lax.fori_loop(0, n_blk // NB, super_blk, 0)

    for b in range(NB):
        pltpu.make_async_copy(
            rows_v.at[b], out_hbm.at[pl.ds(base, R)], wsems[b]
        ).wait()


def kernel(cls_ids, bboxes, cls_embed_table):
    B, T = cls_ids.shape
    V, Dm = cls_embed_table.shape
    ids = cls_ids.reshape(-1).astype(jnp.int32)
    bbox_flat = bboxes.reshape(-1)
    table_pad = jnp.pad(cls_embed_table, ((0, 0), (0, D_OUT - Dm)))

    mesh = plsc.VectorSubcoreMesh(
        core_axis_name="c", subcore_axis_name="s", num_cores=NC, num_subcores=NS
    )
    out = pl.kernel(
        _body,
        out_type=jax.ShapeDtypeStruct((B * T, D_OUT), jnp.float32),
        mesh=mesh,
        compiler_params=pltpu.CompilerParams(use_tc_tiling_on_sc=False),
        scratch_types=[
            pltpu.VMEM((NB, R), jnp.int32),
            pltpu.VMEM((NB, R, D_OUT), jnp.float32),
            pltpu.VMEM((NB, R * 4), jnp.float32),
        ] + [pltpu.SemaphoreType.DMA] * (2 * NB),
    )(ids, bbox_flat, table_pad)
    return out.reshape(B, T, D_OUT)


# speedup vs baseline: 2.2421x; 1.0052x over previous
import jax
import jax.numpy as jnp
from jax import lax
from jax.experimental import pallas as pl
from jax.experimental.pallas import tpu as pltpu
from jax.experimental.pallas import tpu_sc as plsc

NC, NS, L = 2, 16, 16
NW = NC * NS
D_EMB = 124
D_OUT = 128
R = 128
NB = 4


def _body(ids_hbm, bbox_hbm, table_hbm, out_hbm, idx_v, rows_v, bbox_v, *sems):
    gsems, wsems = sems[:NB], sems[NB:]
    wid = lax.axis_index("s") * NC + lax.axis_index("c")
    n_tok = ids_hbm.shape[0]
    per_w = n_tok // NW
    n_blk = per_w // R
    base = wid * per_w

    iota = lax.iota(jnp.int32, L)
    hi_mask = iota >= (L - 4)

    def merge(b):
        def grp(g, c):
            vb = bbox_v[b, pl.ds(g * L, L)]
            for k in range(4):
                row = g * 4 + k
                old = rows_v[b, row, pl.ds(D_OUT - L, L)]
                perm = vb.at[jnp.where(hi_mask, iota - (L - 4) + 4 * k, 0)].get(
                    mode="promise_in_bounds")
                rows_v[b, row, pl.ds(D_OUT - L, L)] = jnp.where(hi_mask, perm, old)
            return c
        lax.fori_loop(0, R // 4, grp, 0, unroll=4)

    def launch(blk_id, b):
        pltpu.sync_copy(ids_hbm.at[pl.ds(base + blk_id * R, R)], idx_v.at[b])
        pltpu.async_copy(table_hbm.at[idx_v.at[b]], rows_v.at[b], gsems[b])

    launch(0, 0)
    launch(1, 1)

    def super_blk(i, carry):
        for b in range(NB):
            t = i * NB + b
            b2 = (b + 2) % NB
            @pl.when(t + 2 < n_blk)
            def _():
                @pl.when(t >= 2)
                def _():
                    pltpu.make_async_copy(
                        rows_v.at[b2], out_hbm.at[pl.ds(base, R)], wsems[b2]
                    ).wait()
                launch(t + 2, b2)
            r0 = base + t * R
            pltpu.sync_copy(bbox_hbm.at[pl.ds(r0 * 4, R * 4)], bbox_v.at[b])
            pltpu.make_async_copy(
                table_hbm.at[idx_v.at[b]], rows_v.at[b], gsems[b]
            ).wait()
            merge(b)
            pltpu.async_copy(rows_v.at[b], out_hbm.at[pl.ds(r0, R)], wsems[b])
        return carry

    lax.fori_loop(0, n_blk // NB, super_blk, 0)

    for b in range(NB):
        pltpu.make_async_copy(
            rows_v.at[b], out_hbm.at[pl.ds(base, R)], wsems[b]
        ).wait()


def _pad_body(x_ref, o_ref):
    o_ref[...] = jnp.pad(x_ref[...], ((0, 0), (0, D_OUT - D_EMB)))


def _pad_table(table):
    V = table.shape[0]
    rows = 1000
    return pl.pallas_call(
        _pad_body,
        grid=(V // rows,),
        in_specs=[pl.BlockSpec((rows, D_EMB), lambda i: (i, 0))],
        out_specs=pl.BlockSpec((rows, D_OUT), lambda i: (i, 0)),
        out_shape=jax.ShapeDtypeStruct((V, D_OUT), jnp.float32),
    )(table)


def kernel(cls_ids, bboxes, cls_embed_table):
    B, T = cls_ids.shape
    V, Dm = cls_embed_table.shape
    ids = cls_ids.reshape(-1).astype(jnp.int32)
    bbox_flat = bboxes.reshape(-1)
    table_pad = _pad_table(cls_embed_table)

    mesh = plsc.VectorSubcoreMesh(
        core_axis_name="c", subcore_axis_name="s", num_cores=NC, num_subcores=NS
    )
    out = pl.kernel(
        _body,
        out_type=jax.ShapeDtypeStruct((B * T, D_OUT), jnp.float32),
        mesh=mesh,
        compiler_params=pltpu.CompilerParams(use_tc_tiling_on_sc=False),
        scratch_types=[
            pltpu.VMEM((NB, R), jnp.int32),
            pltpu.VMEM((NB, R, D_OUT), jnp.float32),
            pltpu.VMEM((NB, R * 4), jnp.float32),
        ] + [pltpu.SemaphoreType.DMA] * (2 * NB),
    )(ids, bbox_flat, table_pad)
    return out.reshape(B, T, D_OUT)


# speedup vs baseline: 4.9181x; 2.1935x over previous
import jax
import jax.numpy as jnp
from jax import lax
from jax.experimental import pallas as pl
from jax.experimental.pallas import tpu as pltpu
from jax.experimental.pallas import tpu_sc as plsc

NC, NS, L = 2, 16, 16
NW = NC * NS
D_EMB = 124
D_OUT = 128
R = 128
NB = 4


def _body(ids_hbm, bbox_hbm, table_hbm, out_hbm,
          idx_v, rows_v, bbox_v, oidx_v, pat_v, *sems):
    gsems, wsems = sems[:NB], sems[NB:]
    w = lax.axis_index("s") * NC + lax.axis_index("c")
    n_t = ids_hbm.shape[0] // NW

    iota = lax.iota(jnp.int32, L)

    for k in range(R // L):
        pat_v[pl.ds(k * L, L)] = (iota + k * L) * n_t

    hi_mask = iota >= (L - 4)
    row_sel = jnp.where(hi_mask, iota - (L - 4), 0)

    def merge(b):
        bb = bbox_v.at[b]

        def one(bi, carry):
            old = rows_v[b, bi, pl.ds(D_OUT - L, L)]
            col = jnp.broadcast_to(bi, (L,)).astype(jnp.int32)
            vals = plsc.load_gather(bb, [row_sel, col])
            rows_v[b, bi, pl.ds(D_OUT - L, L)] = jnp.where(hi_mask, vals, old)
            return carry

        lax.fori_loop(0, R, one, 0, unroll=8)

    def launch(t, b):
        pltpu.sync_copy(ids_hbm.at[t * NW + w], idx_v.at[b])
        pltpu.async_copy(table_hbm.at[idx_v.at[b]], rows_v.at[b], gsems[b])

    launch(0, 0)
    launch(1, 1)

    def super_blk(i, carry):
        for b in range(NB):
            t = i * NB + b
            b2 = (b + 2) % NB
            @pl.when(t + 2 < n_t)
            def _():
                @pl.when(t >= 2)
                def _():
                    pltpu.make_async_copy(
                        rows_v.at[b2], out_hbm.at[oidx_v.at[b2]], wsems[b2]
                    ).wait()
                launch(t + 2, b2)
            pltpu.sync_copy(bbox_hbm.at[pl.ds(t * NW * 4 + w * 4, 4)],
                            bbox_v.at[b])
            base = w * (R * n_t) + t
            for k in range(R // L):
                oidx_v[b, pl.ds(k * L, L)] = pat_v[pl.ds(k * L, L)] + base
            pltpu.make_async_copy(
                table_hbm.at[idx_v.at[b]], rows_v.at[b], gsems[b]
            ).wait()
            merge(b)
            pltpu.async_copy(rows_v.at[b], out_hbm.at[oidx_v.at[b]], wsems[b])
        return carry

    lax.fori_loop(0, n_t // NB, super_blk, 0)

    for b in range(NB):
        pltpu.make_async_copy(
            rows_v.at[b], out_hbm.at[oidx_v.at[b]], wsems[b]
        ).wait()


def _pad_body(x_ref, o_ref):
    o_ref[...] = jnp.pad(x_ref[...], ((0, 0), (0, D_OUT - D_EMB)))


def _pad_table(table):
    V = table.shape[0]
    rows = 1000
    return pl.pallas_call(
        _pad_body,
        grid=(V // rows,),
        in_specs=[pl.BlockSpec((rows, D_EMB), lambda i: (i, 0))],
        out_specs=pl.BlockSpec((rows, D_OUT), lambda i: (i, 0)),
        out_shape=jax.ShapeDtypeStruct((V, D_OUT), jnp.float32),
    )(table)


def kernel(cls_ids, bboxes, cls_embed_table):
    B, T = cls_ids.shape
    V, Dm = cls_embed_table.shape
    n_tok = B * T
    ids_t = cls_ids.astype(jnp.int32).T.reshape(T * B // R, R)
    bbox2 = bboxes.reshape(B // R, R, T, 4).transpose(2, 0, 3, 1).reshape(
        n_tok * 4 // D_OUT, D_OUT)
    table_pad = _pad_table(cls_embed_table)

    mesh = plsc.VectorSubcoreMesh(
        core_axis_name="c", subcore_axis_name="s", num_cores=NC, num_subcores=NS
    )
    out = pl.kernel(
        _body,
        out_type=jax.ShapeDtypeStruct((n_tok, D_OUT), jnp.float32),
        mesh=mesh,
        compiler_params=pltpu.CompilerParams(
            use_tc_tiling_on_sc=True, needs_layout_passes=False),
        scratch_types=[
            pltpu.VMEM((NB, R), jnp.int32),
            pltpu.VMEM((NB, R, D_OUT), jnp.float32),
            pltpu.VMEM((NB, 4, D_OUT), jnp.float32),
            pltpu.VMEM((NB, R), jnp.int32),
            pltpu.VMEM((R,), jnp.int32),
        ] + [pltpu.SemaphoreType.DMA] * (2 * NB),
    )(ids_t, bbox2, table_pad)
    return out.reshape(B, T, D_OUT)


# speedup vs baseline: 7.4331x; 1.5114x over previous
import jax
import jax.numpy as jnp
from jax import lax
from jax.experimental import pallas as pl
from jax.experimental.pallas import tpu as pltpu
from jax.experimental.pallas import tpu_sc as plsc

NC, NS, L = 2, 16, 16
NW = NC * NS
D_EMB = 124
D_OUT = 128
R = 128
NB = 4


def _body(ids_hbm, bbox_hbm, table_hbm, out_hbm,
          idx_v, rows_v, bbox_v, oidx_v, pat_v, *sems):
    gsems, wsems = sems[:NB], sems[NB:]
    w = lax.axis_index("s") * NC + lax.axis_index("c")
    n_t = ids_hbm.shape[0] // NW

    iota = lax.iota(jnp.int32, L)

    for k in range(R // L):
        pat_v[pl.ds(k * L, L)] = (iota + k * L) * n_t

    def merge(b):
        rr = rows_v.at[b]
        for c in range(4):
            col = jnp.full((L,), D_OUT - 4 + c, jnp.int32)
            for k in range(R // L):
                vb = bbox_v[b, c, pl.ds(k * L, L)]
                plsc.store_scatter(rr, [iota + k * L, col], vb)

    def launch(t, b):
        pltpu.sync_copy(ids_hbm.at[t * NW + w], idx_v.at[b])
        pltpu.async_copy(table_hbm.at[idx_v.at[b]], rows_v.at[b], gsems[b])

    launch(0, 0)
    launch(1, 1)

    def super_blk(i, carry):
        for b in range(NB):
            t = i * NB + b
            b2 = (b + 2) % NB
            @pl.when(t + 2 < n_t)
            def _():
                @pl.when(t >= 2)
                def _():
                    pltpu.make_async_copy(
                        rows_v.at[b2], out_hbm.at[oidx_v.at[b2]], wsems[b2]
                    ).wait()
                launch(t + 2, b2)
            pltpu.sync_copy(bbox_hbm.at[pl.ds(t * NW * 4 + w * 4, 4)],
                            bbox_v.at[b])
            base = w * (R * n_t) + t
            for k in range(R // L):
                oidx_v[b, pl.ds(k * L, L)] = pat_v[pl.ds(k * L, L)] + base
            pltpu.make_async_copy(
                table_hbm.at[idx_v.at[b]], rows_v.at[b], gsems[b]
            ).wait()
            merge(b)
            pltpu.async_copy(rows_v.at[b], out_hbm.at[oidx_v.at[b]], wsems[b])
        return carry

    lax.fori_loop(0, n_t // NB, super_blk, 0)

    for b in range(NB):
        pltpu.make_async_copy(
            rows_v.at[b], out_hbm.at[oidx_v.at[b]], wsems[b]
        ).wait()


def _pad_body(x_ref, o_ref):
    o_ref[...] = jnp.pad(x_ref[...], ((0, 0), (0, D_OUT - D_EMB)))


def _pad_table(table):
    V = table.shape[0]
    rows = 1000
    return pl.pallas_call(
        _pad_body,
        grid=(V // rows,),
        in_specs=[pl.BlockSpec((rows, D_EMB), lambda i: (i, 0))],
        out_specs=pl.BlockSpec((rows, D_OUT), lambda i: (i, 0)),
        out_shape=jax.ShapeDtypeStruct((V, D_OUT), jnp.float32),
    )(table)


def kernel(cls_ids, bboxes, cls_embed_table):
    B, T = cls_ids.shape
    V, Dm = cls_embed_table.shape
    n_tok = B * T
    ids_t = cls_ids.astype(jnp.int32).T.reshape(T * B // R, R)
    bbox2 = bboxes.reshape(B // R, R, T, 4).transpose(2, 0, 3, 1).reshape(
        n_tok * 4 // D_OUT, D_OUT)
    table_pad = _pad_table(cls_embed_table)

    mesh = plsc.VectorSubcoreMesh(
        core_axis_name="c", subcore_axis_name="s", num_cores=NC, num_subcores=NS
    )
    out = pl.kernel(
        _body,
        out_type=jax.ShapeDtypeStruct((n_tok, D_OUT), jnp.float32),
        mesh=mesh,
        compiler_params=pltpu.CompilerParams(
            use_tc_tiling_on_sc=True, needs_layout_passes=False),
        scratch_types=[
            pltpu.VMEM((NB, R), jnp.int32),
            pltpu.VMEM((NB, R, D_OUT), jnp.float32),
            pltpu.VMEM((NB, 4, D_OUT), jnp.float32),
            pltpu.VMEM((NB, R), jnp.int32),
            pltpu.VMEM((R,), jnp.int32),
        ] + [pltpu.SemaphoreType.DMA] * (2 * NB),
    )(ids_t, bbox2, table_pad)
    return out.reshape(B, T, D_OUT)


# speedup vs baseline: 8.7693x; 1.1798x over previous
import jax
import jax.numpy as jnp
from jax import lax
from jax.experimental import pallas as pl
from jax.experimental.pallas import tpu as pltpu
from jax.experimental.pallas import tpu_sc as plsc

NC, NS, L = 2, 16, 16
NW = NC * NS
D_EMB = 124
D_OUT = 128
R = 128
NB = 4


def _body(ids_hbm, bbox_hbm, table_hbm, out_hbm,
          idx_all, rows_v, bbox_v, oidx_v, pat_v, *sems):
    gsems, wsems = sems[:NB], sems[NB:]
    w = lax.axis_index("s") * NC + lax.axis_index("c")
    n_t = ids_hbm.shape[0] // NW

    iota = lax.iota(jnp.int32, L)

    pltpu.sync_copy(ids_hbm.at[pl.ds(w * n_t, n_t)], idx_all)

    for k in range(R // L):
        pat_v[pl.ds(k * L, L)] = (iota + k * L) * n_t

    def merge(b):
        rr = rows_v.at[b]
        for c in range(4):
            col = jnp.full((L,), D_OUT - 4 + c, jnp.int32)
            for k in range(R // L):
                vb = bbox_v[b, c, pl.ds(k * L, L)]
                plsc.store_scatter(rr, [iota + k * L, col], vb)

    def launch(t, b):
        pltpu.async_copy(table_hbm.at[idx_all.at[t]], rows_v.at[b], gsems[b])
        pltpu.async_copy(bbox_hbm.at[pl.ds(t * NW * 4 + w * 4, 4)],
                         bbox_v.at[b], gsems[b])

    def wait_launch(t, b):
        pltpu.make_async_copy(
            table_hbm.at[idx_all.at[t]], rows_v.at[b], gsems[b]).wait()
        pltpu.make_async_copy(
            bbox_hbm.at[pl.ds(0, 4)], bbox_v.at[b], gsems[b]).wait()

    launch(0, 0)
    launch(1, 1)

    def super_blk(i, carry):
        for b in range(NB):
            t = i * NB + b
            b2 = (b + 2) % NB
            @pl.when(t + 2 < n_t)
            def _():
                @pl.when(t >= 2)
                def _():
                    pltpu.make_async_copy(
                        rows_v.at[b2], out_hbm.at[oidx_v.at[b2]], wsems[b2]
                    ).wait()
                launch(t + 2, b2)
            base = w * (R * n_t) + t
            for k in range(R // L):
                oidx_v[b, pl.ds(k * L, L)] = pat_v[pl.ds(k * L, L)] + base
            wait_launch(t, b)
            merge(b)
            pltpu.async_copy(rows_v.at[b], out_hbm.at[oidx_v.at[b]], wsems[b])
        return carry

    lax.fori_loop(0, n_t // NB, super_blk, 0)

    for b in range(NB):
        pltpu.make_async_copy(
            rows_v.at[b], out_hbm.at[oidx_v.at[b]], wsems[b]
        ).wait()


def _pad_body(x_ref, o_ref):
    o_ref[...] = jnp.pad(x_ref[...], ((0, 0), (0, D_OUT - D_EMB)))


def _pad_table(table):
    V = table.shape[0]
    rows = 1000
    return pl.pallas_call(
        _pad_body,
        grid=(V // rows,),
        in_specs=[pl.BlockSpec((rows, D_EMB), lambda i: (i, 0))],
        out_specs=pl.BlockSpec((rows, D_OUT), lambda i: (i, 0)),
        out_shape=jax.ShapeDtypeStruct((V, D_OUT), jnp.float32),
    )(table)


def kernel(cls_ids, bboxes, cls_embed_table):
    B, T = cls_ids.shape
    V, Dm = cls_embed_table.shape
    n_tok = B * T
    ids_t = (cls_ids.astype(jnp.int32).T.reshape(T, B // R, R)
             .transpose(1, 0, 2).reshape(T * B // R, R))
    bbox2 = bboxes.reshape(B // R, R, T, 4).transpose(2, 0, 3, 1).reshape(
        n_tok * 4 // D_OUT, D_OUT)
    table_pad = _pad_table(cls_embed_table)

    mesh = plsc.VectorSubcoreMesh(
        core_axis_name="c", subcore_axis_name="s", num_cores=NC, num_subcores=NS
    )
    out = pl.kernel(
        _body,
        out_type=jax.ShapeDtypeStruct((n_tok, D_OUT), jnp.float32),
        mesh=mesh,
        compiler_params=pltpu.CompilerParams(
            use_tc_tiling_on_sc=True, needs_layout_passes=False),
        scratch_types=[
            pltpu.VMEM((T, R), jnp.int32),
            pltpu.VMEM((NB, R, D_OUT), jnp.float32),
            pltpu.VMEM((NB, 4, D_OUT), jnp.float32),
            pltpu.VMEM((NB, R), jnp.int32),
            pltpu.VMEM((R,), jnp.int32),
        ] + [pltpu.SemaphoreType.DMA] * (2 * NB),
    )(ids_t, bbox2, table_pad)
    return out.reshape(B, T, D_OUT)


# speedup vs baseline: 8.8092x; 1.0045x over previous
import jax
import jax.numpy as jnp
from jax import lax
from jax.experimental import pallas as pl
from jax.experimental.pallas import tpu as pltpu
from jax.experimental.pallas import tpu_sc as plsc

NC, NS, L = 2, 16, 16
NW = NC * NS
D_EMB = 124
D_OUT = 128
R = 128
NB = 5
LEAD = 3


def _body(ids_hbm, bbox_hbm, table_hbm, out_hbm,
          idx_all, rows_v, bbox_v, oidx_v, pat_v, *sems):
    gsems, wsems = sems[:NB], sems[NB:]
    w = lax.axis_index("s") * NC + lax.axis_index("c")
    n_t = ids_hbm.shape[0] // NW

    iota = lax.iota(jnp.int32, L)

    pltpu.sync_copy(ids_hbm.at[pl.ds(w * n_t, n_t)], idx_all)

    for k in range(R // L):
        pat_v[pl.ds(k * L, L)] = (iota + k * L) * n_t

    def merge(b):
        rr = rows_v.at[b]
        for c in range(4):
            col = jnp.full((L,), D_OUT - 4 + c, jnp.int32)
            for k in range(R // L):
                vb = bbox_v[b, c, pl.ds(k * L, L)]
                plsc.store_scatter(rr, [iota + k * L, col], vb)

    def launch(t, b):
        pltpu.async_copy(table_hbm.at[idx_all.at[t]], rows_v.at[b], gsems[b])
        pltpu.async_copy(bbox_hbm.at[pl.ds(t * NW * 4 + w * 4, 4)],
                         bbox_v.at[b], gsems[b])

    def wait_launch(t, b):
        pltpu.make_async_copy(
            table_hbm.at[idx_all.at[t]], rows_v.at[b], gsems[b]).wait()
        pltpu.make_async_copy(
            bbox_hbm.at[pl.ds(0, 4)], bbox_v.at[b], gsems[b]).wait()

    for p in range(LEAD):
        launch(p, p)

    def super_blk(i, carry):
        for b in range(NB):
            t = i * NB + b
            b2 = (b + LEAD) % NB
            @pl.when(t + LEAD < n_t)
            def _():
                @pl.when(t >= NB - LEAD)
                def _():
                    pltpu.make_async_copy(
                        rows_v.at[b2], out_hbm.at[oidx_v.at[b2]], wsems[b2]
                    ).wait()
                launch(t + LEAD, b2)
            base = w * (R * n_t) + t
            for k in range(R // L):
                oidx_v[b, pl.ds(k * L, L)] = pat_v[pl.ds(k * L, L)] + base
            wait_launch(t, b)
            merge(b)
            pltpu.async_copy(rows_v.at[b], out_hbm.at[oidx_v.at[b]], wsems[b])
        return carry

    lax.fori_loop(0, n_t // NB, super_blk, 0)

    for b in range(NB):
        pltpu.make_async_copy(
            rows_v.at[b], out_hbm.at[oidx_v.at[b]], wsems[b]
        ).wait()


def _pad_body(x_ref, o_ref):
    o_ref[...] = jnp.pad(x_ref[...], ((0, 0), (0, D_OUT - D_EMB)))


def _pad_table(table):
    V = table.shape[0]
    rows = 1000
    return pl.pallas_call(
        _pad_body,
        grid=(V // rows,),
        in_specs=[pl.BlockSpec((rows, D_EMB), lambda i: (i, 0))],
        out_specs=pl.BlockSpec((rows, D_OUT), lambda i: (i, 0)),
        out_shape=jax.ShapeDtypeStruct((V, D_OUT), jnp.float32),
    )(table)


def kernel(cls_ids, bboxes, cls_embed_table):
    B, T = cls_ids.shape
    V, Dm = cls_embed_table.shape
    n_tok = B * T
    ids_t = (cls_ids.astype(jnp.int32).T.reshape(T, B // R, R)
             .transpose(1, 0, 2).reshape(T * B // R, R))
    bbox2 = bboxes.reshape(B // R, R, T, 4).transpose(2, 0, 3, 1).reshape(
        n_tok * 4 // D_OUT, D_OUT)
    table_pad = _pad_table(cls_embed_table)

    mesh = plsc.VectorSubcoreMesh(
        core_axis_name="c", subcore_axis_name="s", num_cores=NC, num_subcores=NS
    )
    out = pl.kernel(
        _body,
        out_type=jax.ShapeDtypeStruct((n_tok, D_OUT), jnp.float32),
        mesh=mesh,
        compiler_params=pltpu.CompilerParams(
            use_tc_tiling_on_sc=True, needs_layout_passes=False),
        scratch_types=[
            pltpu.VMEM((T, R), jnp.int32),
            pltpu.VMEM((NB, R, D_OUT), jnp.float32),
            pltpu.VMEM((NB, 4, D_OUT), jnp.float32),
            pltpu.VMEM((NB, R), jnp.int32),
            pltpu.VMEM((R,), jnp.int32),
        ] + [pltpu.SemaphoreType.DMA] * (2 * NB),
    )(ids_t, bbox2, table_pad)
    return out.reshape(B, T, D_OUT)
